# baseline (speedup 1.0000x reference)
import jax
import jax.numpy as jnp
from jax import lax
from jax.experimental import pallas as pl
from jax.experimental.pallas import tpu as pltpu
from jax.experimental.pallas import tpu_sc as plsc

_N = 10000
_E = 320000
_DH = 64
_NC = 2
_NS = 16
_NW = _NC * _NS
_EPT = _E // _NW
_C = 80
_NCH = _EPT // _C
_NB = 8
_RPT = _N // _NS
_ZR = 125


def _vmesh():
    return plsc.VectorSubcoreMesh(
        core_axis_name="c", subcore_axis_name="s",
        num_cores=_NC, num_subcores=_NS)


def _deg_body(ei_hbm, out_hbm, dst_v, deg_v, dsem):
    cid = lax.axis_index("c")
    sid = lax.axis_index("s")
    wid = sid * _NC + cid
    d = pltpu.async_copy(ei_hbm.at[1, pl.ds(wid * _EPT, _EPT)], dst_v, dsem)
    zeros16 = jnp.zeros((16,), jnp.float32)

    def _zero(i, _):
        deg_v[pl.ds(i * 16, 16)] = zeros16
        return 0

    lax.fori_loop(0, _N // 16, _zero, 0, unroll=4)
    d.wait()

    def _hist(i, _):
        idx = dst_v[pl.ds(i * 16, 16)]
        cnt, last = plsc.scan_count(idx)
        plsc.addupdate_scatter(
            deg_v, [idx], cnt.astype(jnp.float32), mask=last)
        return 0

    lax.fori_loop(0, _EPT // 16, _hist, 0, unroll=4)
    pltpu.sync_copy(deg_v, out_hbm.at[wid])


_deg_call = pl.kernel(
    _deg_body,
    out_type=jax.ShapeDtypeStruct((_NW, _N), jnp.float32),
    mesh=_vmesh(),
    scratch_types=[
        pltpu.VMEM((_EPT,), jnp.int32),
        pltpu.VMEM((_N,), jnp.float32),
        pltpu.SemaphoreType.DMA,
    ],
    compiler_params=pltpu.CompilerParams(
        needs_layout_passes=False, use_tc_tiling_on_sc=False),
)


def _gs_body(y_hbm, ei_hbm, out_hbm, src_v, dst_f, dst_v, rows_v, zbuf,
             acc, gsem, ssem, isem):
    cid = lax.axis_index("c")
    sid = lax.axis_index("s")
    wid = sid * _NC + cid

    d1 = pltpu.async_copy(ei_hbm.at[0, pl.ds(wid * _EPT, _EPT)], src_v, isem)
    d2 = pltpu.async_copy(ei_hbm.at[1, pl.ds(wid * _EPT, _EPT)], dst_f, isem)
    d1.wait()

    for b in range(_NB - 1):
        pltpu.async_copy(y_hbm.at[src_v.at[pl.ds(b * _C, _C)]],
                         rows_v.at[b], gsem.at[b])

    zeros16 = jnp.zeros((16,), jnp.float32)

    def _zrow(i, _):
        for j in range(_DH // 16):
            zbuf[i, pl.ds(j * 16, 16)] = zeros16
        return 0

    lax.fori_loop(0, _ZR, _zrow, 0)

    def _zcp(k, _):
        pltpu.sync_copy(zbuf, acc.at[pl.ds(sid * _RPT + k * _ZR, _ZR)])
        return 0

    lax.fori_loop(0, _RPT // _ZR, _zcp, 0)
    d2.wait()

    def _dcp(i, _):
        v = dst_f[pl.ds(i * 16, 16)]
        dst_v[lax.div(i, _C // 16), pl.ds(lax.rem(i, _C // 16) * 16, 16)] = v
        return 0

    lax.fori_loop(0, _EPT // 16, _dcp, 0, unroll=4)
    plsc.subcore_barrier()


    def _chunk(j, _):
        b = lax.rem(j, _NB)
        pb = lax.rem(j + _NB - 1, _NB)
        pltpu.make_async_copy(
            y_hbm.at[src_v.at[pl.ds(0, _C)]], rows_v.at[b],
            gsem.at[b]).wait()
        pltpu.async_copy(rows_v.at[b], acc.at[dst_v.at[j]], ssem.at[b],
                         add=True)

        @pl.when(j > 0)
        def _wait_prev_scatter():
            pltpu.make_async_copy(rows_v.at[pb], acc.at[dst_v.at[0]],
                                  ssem.at[pb]).wait()

        @pl.when(j < _NCH - (_NB - 1))
        def _start_next_gather():
            pltpu.async_copy(
                y_hbm.at[src_v.at[pl.ds((j + _NB - 1) * _C, _C)]],
                rows_v.at[pb], gsem.at[pb])

        return 0

    lax.fori_loop(0, _NCH, _chunk, 0)
    pltpu.make_async_copy(rows_v.at[(_NCH - 1) % _NB], acc.at[dst_v.at[0]],
                          ssem.at[(_NCH - 1) % _NB]).wait()
    plsc.subcore_barrier()

    pltpu.sync_copy(acc.at[pl.ds(sid * _RPT, _RPT)],
                    out_hbm.at[pl.ds(cid * _N + sid * _RPT, _RPT)])


_gs_call = pl.kernel(
    _gs_body,
    out_type=jax.ShapeDtypeStruct((_NC * _N, _DH), jnp.float32),
    mesh=_vmesh(),
    scratch_types=[
        pltpu.VMEM((_EPT,), jnp.int32),
        pltpu.VMEM((_EPT,), jnp.int32),
        pltpu.VMEM((_NCH, _C), jnp.int32),
        pltpu.VMEM((_NB, _C, _DH), jnp.float32),
        pltpu.VMEM((_ZR, _DH), jnp.float32),
        pltpu.VMEM_SHARED((_N, _DH), jnp.float32),
        pltpu.SemaphoreType.DMA((_NB,)),
        pltpu.SemaphoreType.DMA((_NB,)),
        pltpu.SemaphoreType.DMA,
    ],
    compiler_params=pltpu.CompilerParams(use_tc_tiling_on_sc=False),
)


def _t1_body(x_ref, w_ref, degp_ref, y_ref, dinv_ref):
    deg = jnp.sum(degp_ref[...], axis=0) + 1.0
    dinv = lax.rsqrt(jnp.maximum(deg, 1e-12))
    dcol = dinv[:, None]
    xl = jnp.dot(x_ref[...], w_ref[...], preferred_element_type=jnp.float32)
    y_ref[...] = xl * dcol
    dinv_ref[...] = dcol


_t1_call = pl.pallas_call(
    _t1_body,
    out_shape=(
        jax.ShapeDtypeStruct((_N, _DH), jnp.float32),
        jax.ShapeDtypeStruct((_N, 1), jnp.float32),
    ),
)


def _t2_body(aggp_ref, y_ref, dinv_ref, b_ref, w_ref, y2_ref):
    dinv = dinv_ref[...]
    agg = aggp_ref[0:_N, :] + aggp_ref[_N:2 * _N, :] + y_ref[...]
    h = jnp.maximum(agg * dinv + b_ref[...][None, :], 0.0)
    y2_ref[...] = jnp.dot(
        h, w_ref[...], preferred_element_type=jnp.float32) * dinv


_t2_call = pl.pallas_call(
    _t2_body,
    out_shape=jax.ShapeDtypeStruct((_N, _DH), jnp.float32),
)


def _t3_body(aggp_ref, y_ref, dinv_ref, b_ref, out_ref):
    agg = aggp_ref[0:_N, :] + aggp_ref[_N:2 * _N, :] + y_ref[...]
    out_ref[...] = jnp.maximum(
        agg * dinv_ref[...] + b_ref[...][None, :], 0.0)


_t3_call = pl.pallas_call(
    _t3_body,
    out_shape=jax.ShapeDtypeStruct((_N, _DH), jnp.float32),
)


def kernel(x, edge_index, W1, b1, W2, b2):
    degp = _deg_call(edge_index)
    y1, dinv = _t1_call(x, W1, degp)
    agg1 = _gs_call(y1, edge_index)
    y2 = _t2_call(agg1, y1, dinv, b1, W2)
    agg2 = _gs_call(y2, edge_index)
    h = _t3_call(agg2, y2, dinv, b2)
    return h


# speedup vs baseline: 1.0052x; 1.0052x over previous
import jax
import jax.numpy as jnp
from jax import lax
from jax.experimental import pallas as pl
from jax.experimental.pallas import tpu as pltpu
from jax.experimental.pallas import tpu_sc as plsc

_N = 10000
_E = 320000
_DH = 64
_NC = 2
_NS = 16
_NW = _NC * _NS
_EPT = _E // _NW
_C = 80
_NCH = _EPT // _C
_NB = 8
_RPT = _N // _NS
_ZR = 125


def _vmesh():
    return plsc.VectorSubcoreMesh(
        core_axis_name="c", subcore_axis_name="s",
        num_cores=_NC, num_subcores=_NS)


def _deg_body(ei_hbm, out_hbm, dst_v, deg_v, dsem):
    cid = lax.axis_index("c")
    sid = lax.axis_index("s")
    wid = sid * _NC + cid
    d = pltpu.async_copy(ei_hbm.at[1, pl.ds(wid * _EPT, _EPT)], dst_v, dsem)
    zeros16 = jnp.zeros((16,), jnp.float32)

    def _zero(i, _):
        deg_v[pl.ds(i * 16, 16)] = zeros16
        return 0

    lax.fori_loop(0, _N // 16, _zero, 0, unroll=4)
    d.wait()

    def _hist(i, _):
        idx = dst_v[pl.ds(i * 16, 16)]
        cnt, last = plsc.scan_count(idx)
        plsc.addupdate_scatter(
            deg_v, [idx], cnt.astype(jnp.float32), mask=last)
        return 0

    lax.fori_loop(0, _EPT // 16, _hist, 0, unroll=4)
    pltpu.sync_copy(deg_v, out_hbm.at[wid])


_deg_call = pl.kernel(
    _deg_body,
    out_type=jax.ShapeDtypeStruct((_NW, _N), jnp.float32),
    mesh=_vmesh(),
    scratch_types=[
        pltpu.VMEM((_EPT,), jnp.int32),
        pltpu.VMEM((_N,), jnp.float32),
        pltpu.SemaphoreType.DMA,
    ],
    compiler_params=pltpu.CompilerParams(
        needs_layout_passes=False, use_tc_tiling_on_sc=False),
)


def _gs_body(y_hbm, ei_hbm, dst_hbm, out_hbm, src_v, dst_v, rows_v, zbuf,
             acc, gsem, ssem, isem):
    cid = lax.axis_index("c")
    sid = lax.axis_index("s")
    wid = sid * _NC + cid

    d1 = pltpu.async_copy(ei_hbm.at[0, pl.ds(wid * _EPT, _EPT)], src_v, isem)
    d2 = pltpu.async_copy(dst_hbm.at[wid], dst_v, isem)
    d1.wait()

    for b in range(_NB - 1):
        pltpu.async_copy(y_hbm.at[src_v.at[pl.ds(b * _C, _C)]],
                         rows_v.at[b], gsem.at[b])

    zeros16 = jnp.zeros((16,), jnp.float32)

    def _zrow(i, _):
        for j in range(_DH // 16):
            zbuf[i, pl.ds(j * 16, 16)] = zeros16
        return 0

    lax.fori_loop(0, _ZR, _zrow, 0)

    def _zcp(k, _):
        pltpu.sync_copy(zbuf, acc.at[pl.ds(sid * _RPT + k * _ZR, _ZR)])
        return 0

    lax.fori_loop(0, _RPT // _ZR, _zcp, 0)
    d2.wait()
    plsc.subcore_barrier()


    def _chunk(j, _):
        b = lax.rem(j, _NB)
        pb = lax.rem(j + _NB - 1, _NB)
        pltpu.make_async_copy(
            y_hbm.at[src_v.at[pl.ds(0, _C)]], rows_v.at[b],
            gsem.at[b]).wait()
        pltpu.async_copy(rows_v.at[b], acc.at[dst_v.at[j]], ssem.at[b],
                         add=True)

        @pl.when(j > 0)
        def _wait_prev_scatter():
            pltpu.make_async_copy(rows_v.at[pb], acc.at[dst_v.at[0]],
                                  ssem.at[pb]).wait()

        @pl.when(j < _NCH - (_NB - 1))
        def _start_next_gather():
            pltpu.async_copy(
                y_hbm.at[src_v.at[pl.ds((j + _NB - 1) * _C, _C)]],
                rows_v.at[pb], gsem.at[pb])

        return 0

    lax.fori_loop(0, _NCH, _chunk, 0)
    pltpu.make_async_copy(rows_v.at[(_NCH - 1) % _NB], acc.at[dst_v.at[0]],
                          ssem.at[(_NCH - 1) % _NB]).wait()
    plsc.subcore_barrier()

    pltpu.sync_copy(acc.at[pl.ds(sid * _RPT, _RPT)],
                    out_hbm.at[pl.ds(cid * _N + sid * _RPT, _RPT)])


_gs_call = pl.kernel(
    _gs_body,
    out_type=jax.ShapeDtypeStruct((_NC * _N, _DH), jnp.float32),
    mesh=_vmesh(),
    scratch_types=[
        pltpu.VMEM((_EPT,), jnp.int32),
        pltpu.VMEM((_NCH, _C), jnp.int32),
        pltpu.VMEM((_NB, _C, _DH), jnp.float32),
        pltpu.VMEM((_ZR, _DH), jnp.float32),
        pltpu.VMEM_SHARED((_N, _DH), jnp.float32),
        pltpu.SemaphoreType.DMA((_NB,)),
        pltpu.SemaphoreType.DMA((_NB,)),
        pltpu.SemaphoreType.DMA,
    ],
    compiler_params=pltpu.CompilerParams(use_tc_tiling_on_sc=False),
)


def _t1_body(x_ref, w_ref, degp_ref, y_ref, dinv_ref):
    deg = jnp.sum(degp_ref[...], axis=0) + 1.0
    dinv = lax.rsqrt(jnp.maximum(deg, 1e-12))
    dcol = dinv[:, None]
    xl = jnp.dot(x_ref[...], w_ref[...], preferred_element_type=jnp.float32)
    y_ref[...] = xl * dcol
    dinv_ref[...] = dcol


_t1_call = pl.pallas_call(
    _t1_body,
    out_shape=(
        jax.ShapeDtypeStruct((_N, _DH), jnp.float32),
        jax.ShapeDtypeStruct((_N, 1), jnp.float32),
    ),
)


def _t2_body(aggp_ref, y_ref, dinv_ref, b_ref, w_ref, y2_ref):
    dinv = dinv_ref[...]
    agg = aggp_ref[0:_N, :] + aggp_ref[_N:2 * _N, :] + y_ref[...]
    h = jnp.maximum(agg * dinv + b_ref[...][None, :], 0.0)
    y2_ref[...] = jnp.dot(
        h, w_ref[...], preferred_element_type=jnp.float32) * dinv


_t2_call = pl.pallas_call(
    _t2_body,
    out_shape=jax.ShapeDtypeStruct((_N, _DH), jnp.float32),
)


def _t3_body(aggp_ref, y_ref, dinv_ref, b_ref, out_ref):
    agg = aggp_ref[0:_N, :] + aggp_ref[_N:2 * _N, :] + y_ref[...]
    out_ref[...] = jnp.maximum(
        agg * dinv_ref[...] + b_ref[...][None, :], 0.0)


_t3_call = pl.pallas_call(
    _t3_body,
    out_shape=jax.ShapeDtypeStruct((_N, _DH), jnp.float32),
)


def kernel(x, edge_index, W1, b1, W2, b2):
    dst = edge_index[1].reshape(_NW, _NCH, _C)

    degp = _deg_call(edge_index)
    y1, dinv = _t1_call(x, W1, degp)
    agg1 = _gs_call(y1, edge_index, dst)
    y2 = _t2_call(agg1, y1, dinv, b1, W2)
    agg2 = _gs_call(y2, edge_index, dst)
    h = _t3_call(agg2, y2, dinv, b2)
    return h


# speedup vs baseline: 1.0070x; 1.0018x over previous
import jax
import jax.numpy as jnp
from jax import lax
from jax.experimental import pallas as pl
from jax.experimental.pallas import tpu as pltpu
from jax.experimental.pallas import tpu_sc as plsc

_N = 10000
_E = 320000
_DH = 64
_NC = 2
_NS = 16
_NW = _NC * _NS
_EPT = _E // _NW
_C = 80
_NCH = _EPT // _C
_NB = 10
_RPT = _N // _NS
_ZR = 125


def _vmesh():
    return plsc.VectorSubcoreMesh(
        core_axis_name="c", subcore_axis_name="s",
        num_cores=_NC, num_subcores=_NS)


def _deg_body(ei_hbm, out_hbm, dst_v, deg_v, dsem):
    cid = lax.axis_index("c")
    sid = lax.axis_index("s")
    wid = sid * _NC + cid
    d = pltpu.async_copy(ei_hbm.at[1, pl.ds(wid * _EPT, _EPT)], dst_v, dsem)
    zeros16 = jnp.zeros((16,), jnp.float32)

    def _zero(i, _):
        deg_v[pl.ds(i * 16, 16)] = zeros16
        return 0

    lax.fori_loop(0, _N // 16, _zero, 0, unroll=4)
    d.wait()

    def _hist(i, _):
        idx = dst_v[pl.ds(i * 16, 16)]
        cnt, last = plsc.scan_count(idx)
        plsc.addupdate_scatter(
            deg_v, [idx], cnt.astype(jnp.float32), mask=last)
        return 0

    lax.fori_loop(0, _EPT // 16, _hist, 0, unroll=4)
    pltpu.sync_copy(deg_v, out_hbm.at[wid])


_deg_call = pl.kernel(
    _deg_body,
    out_type=jax.ShapeDtypeStruct((_NW, _N), jnp.float32),
    mesh=_vmesh(),
    scratch_types=[
        pltpu.VMEM((_EPT,), jnp.int32),
        pltpu.VMEM((_N,), jnp.float32),
        pltpu.SemaphoreType.DMA,
    ],
    compiler_params=pltpu.CompilerParams(
        needs_layout_passes=False, use_tc_tiling_on_sc=False),
)


def _gs_body(y_hbm, ei_hbm, dst_hbm, out_hbm, src_v, dst_v, rows_v, zbuf,
             acc, gsem, ssem, isem):
    cid = lax.axis_index("c")
    sid = lax.axis_index("s")
    wid = sid * _NC + cid

    d1 = pltpu.async_copy(ei_hbm.at[0, pl.ds(wid * _EPT, _EPT)], src_v, isem)
    d2 = pltpu.async_copy(dst_hbm.at[wid], dst_v, isem)
    d1.wait()

    for b in range(_NB - 1):
        pltpu.async_copy(y_hbm.at[src_v.at[pl.ds(b * _C, _C)]],
                         rows_v.at[b], gsem.at[b])

    zeros16 = jnp.zeros((16,), jnp.float32)

    def _zrow(i, _):
        for j in range(_DH // 16):
            zbuf[i, pl.ds(j * 16, 16)] = zeros16
        return 0

    lax.fori_loop(0, _ZR, _zrow, 0)

    def _zcp(k, _):
        pltpu.sync_copy(zbuf, acc.at[pl.ds(sid * _RPT + k * _ZR, _ZR)])
        return 0

    lax.fori_loop(0, _RPT // _ZR, _zcp, 0)
    d2.wait()
    plsc.subcore_barrier()


    def _chunk(j, _):
        b = lax.rem(j, _NB)
        pb = lax.rem(j + _NB - 1, _NB)
        pltpu.make_async_copy(
            y_hbm.at[src_v.at[pl.ds(0, _C)]], rows_v.at[b],
            gsem.at[b]).wait()
        pltpu.async_copy(rows_v.at[b], acc.at[dst_v.at[j]], ssem.at[b],
                         add=True)

        @pl.when(j > 0)
        def _wait_prev_scatter():
            pltpu.make_async_copy(rows_v.at[pb], acc.at[dst_v.at[0]],
                                  ssem.at[pb]).wait()

        @pl.when(j < _NCH - (_NB - 1))
        def _start_next_gather():
            pltpu.async_copy(
                y_hbm.at[src_v.at[pl.ds((j + _NB - 1) * _C, _C)]],
                rows_v.at[pb], gsem.at[pb])

        return 0

    lax.fori_loop(0, _NCH, _chunk, 0)
    pltpu.make_async_copy(rows_v.at[(_NCH - 1) % _NB], acc.at[dst_v.at[0]],
                          ssem.at[(_NCH - 1) % _NB]).wait()
    plsc.subcore_barrier()

    pltpu.sync_copy(acc.at[pl.ds(sid * _RPT, _RPT)],
                    out_hbm.at[pl.ds(cid * _N + sid * _RPT, _RPT)])


_gs_call = pl.kernel(
    _gs_body,
    out_type=jax.ShapeDtypeStruct((_NC * _N, _DH), jnp.float32),
    mesh=_vmesh(),
    scratch_types=[
        pltpu.VMEM((_EPT,), jnp.int32),
        pltpu.VMEM((_NCH, _C), jnp.int32),
        pltpu.VMEM((_NB, _C, _DH), jnp.float32),
        pltpu.VMEM((_ZR, _DH), jnp.float32),
        pltpu.VMEM_SHARED((_N, _DH), jnp.float32),
        pltpu.SemaphoreType.DMA((_NB,)),
        pltpu.SemaphoreType.DMA((_NB,)),
        pltpu.SemaphoreType.DMA,
    ],
    compiler_params=pltpu.CompilerParams(use_tc_tiling_on_sc=False),
)


def _t1_body(x_ref, w_ref, degp_ref, y_ref, dinv_ref):
    deg = jnp.sum(degp_ref[...], axis=0) + 1.0
    dinv = lax.rsqrt(jnp.maximum(deg, 1e-12))
    dcol = dinv[:, None]
    xl = jnp.dot(x_ref[...], w_ref[...], preferred_element_type=jnp.float32)
    y_ref[...] = xl * dcol
    dinv_ref[...] = dcol


_t1_call = pl.pallas_call(
    _t1_body,
    out_shape=(
        jax.ShapeDtypeStruct((_N, _DH), jnp.float32),
        jax.ShapeDtypeStruct((_N, 1), jnp.float32),
    ),
)


def _t2_body(aggp_ref, y_ref, dinv_ref, b_ref, w_ref, y2_ref):
    dinv = dinv_ref[...]
    agg = aggp_ref[0:_N, :] + aggp_ref[_N:2 * _N, :] + y_ref[...]
    h = jnp.maximum(agg * dinv + b_ref[...][None, :], 0.0)
    y2_ref[...] = jnp.dot(
        h, w_ref[...], preferred_element_type=jnp.float32) * dinv


_t2_call = pl.pallas_call(
    _t2_body,
    out_shape=jax.ShapeDtypeStruct((_N, _DH), jnp.float32),
)


def _t3_body(aggp_ref, y_ref, dinv_ref, b_ref, out_ref):
    agg = aggp_ref[0:_N, :] + aggp_ref[_N:2 * _N, :] + y_ref[...]
    out_ref[...] = jnp.maximum(
        agg * dinv_ref[...] + b_ref[...][None, :], 0.0)


_t3_call = pl.pallas_call(
    _t3_body,
    out_shape=jax.ShapeDtypeStruct((_N, _DH), jnp.float32),
)


def kernel(x, edge_index, W1, b1, W2, b2):
    dst = edge_index[1].reshape(_NW, _NCH, _C)

    degp = _deg_call(edge_index)
    y1, dinv = _t1_call(x, W1, degp)
    agg1 = _gs_call(y1, edge_index, dst)
    y2 = _t2_call(agg1, y1, dinv, b1, W2)
    agg2 = _gs_call(y2, edge_index, dst)
    h = _t3_call(agg2, y2, dinv, b2)
    return h
